# Initial kernel scaffold; baseline (speedup 1.0000x reference)
import jax
import jax.numpy as jnp
from jax.experimental import pallas as pl


def kernel(x, params):
    raise NotImplementedError("write your pallas kernel here")


# speedup vs baseline: 2.9857x; 2.9857x over previous
import itertools
import jax
import jax.numpy as jnp
from jax.experimental import pallas as pl

F32 = jnp.float32
BF16 = jnp.bfloat16


def _blockdiag_mats(w, W):
    Co, Ci = w.shape[:2]
    wi = jnp.arange(W)[:, None]
    wo = jnp.arange(W)[None, :]
    mats = []
    for a, b, c, d in itertools.product(range(3), repeat=4):
        sel = (wi - wo + 1 == d).astype(w.dtype)
        M = jnp.einsum('wv,io->wivo', sel, w[:, :, a, b, c, d].T).reshape(W * Ci, W * Co)
        mats.append(M)
    return jnp.stack(mats)


def _down_mats(w, W):
    Co, Ci = w.shape[:2]
    W2 = W // 2
    wi = jnp.arange(W)[:, None]
    wo = jnp.arange(W2)[None, :]
    mats = []
    for a, b, c, d in itertools.product(range(2), repeat=4):
        sel = (wi == 2 * wo + d).astype(w.dtype)
        M = jnp.einsum('wv,io->wivo', sel, w[:, :, a, b, c, d].T).reshape(W * Ci, W2 * Co)
        mats.append(M)
    return jnp.stack(mats)


def _occ_mat(wocc, W, C):
    Co = wocc.shape[0]
    q = wocc[:, :, 0, 0, 0, 0].T
    eye = jnp.eye(W, dtype=q.dtype)
    return jnp.einsum('wv,io->wivo', eye, q).reshape(W * C, W * Co)


def _pallas_conv_same(xs, M, T, D, H, W, Ci, Co):
    R = T * D * H
    RB = D * H
    L = W * Ci
    N = W * Co

    if RB % 8 == 0:
        def kfn(xs_ref, m_ref, o_ref):
            t = pl.program_id(0)
            i = pl.program_id(1)
            a = i // 27
            b = (i // 9) % 3
            c = (i // 3) % 3
            patch = xs_ref[pl.ds(c, 1), pl.ds(t + a, 1), pl.ds(b, D), :, :].reshape(RB, L)
            dd = jnp.dot(patch, m_ref[i], preferred_element_type=F32)

            @pl.when(i == 0)
            def _():
                o_ref[...] = dd

            @pl.when(i > 0)
            def _():
                o_ref[...] = o_ref[...] + dd

        return pl.pallas_call(
            kfn,
            grid=(T, 81),
            in_specs=[pl.BlockSpec((3, T + 2, D + 2, H, L), lambda t, i: (0, 0, 0, 0, 0)),
                      pl.BlockSpec((81, L, N), lambda t, i: (0, 0, 0))],
            out_specs=pl.BlockSpec((RB, N), lambda t, i: (t, 0)),
            out_shape=jax.ShapeDtypeStruct((R, N), F32))(xs, M)

    def kfn_s(xs_ref, m_ref, o_ref):
        i = pl.program_id(0)
        a = i // 27
        b = (i // 9) % 3
        c = (i // 3) % 3
        patch = xs_ref[pl.ds(c, 1), pl.ds(a, T), pl.ds(b, D), :, :].reshape(R, L)
        dd = jnp.dot(patch, m_ref[i], preferred_element_type=F32)

        @pl.when(i == 0)
        def _():
            o_ref[...] = dd

        @pl.when(i > 0)
        def _():
            o_ref[...] = o_ref[...] + dd

    return pl.pallas_call(
        kfn_s,
        grid=(81,),
        in_specs=[pl.BlockSpec((3, T + 2, D + 2, H, L), lambda i: (0, 0, 0, 0, 0)),
                  pl.BlockSpec((81, L, N), lambda i: (0, 0, 0))],
        out_specs=pl.BlockSpec((R, N), lambda i: (0, 0)),
        out_shape=jax.ShapeDtypeStruct((R, N), F32))(xs, M)


def _pallas_conv_down(P, M, R2, W, Ci, Co):
    W2 = W // 2
    L = W * Ci
    N = W2 * Co

    def kfn(p_ref, m_ref, o_ref):
        i = pl.program_id(0)
        patch = p_ref[i // 2]
        dd = jnp.dot(patch, m_ref[i], preferred_element_type=F32)

        @pl.when(i == 0)
        def _():
            o_ref[...] = dd

        @pl.when(i > 0)
        def _():
            o_ref[...] = o_ref[...] + dd

    return pl.pallas_call(
        kfn,
        grid=(16,),
        in_specs=[pl.BlockSpec((8, R2, L), lambda i: (0, 0, 0)),
                  pl.BlockSpec((16, L, N), lambda i: (0, 0, 0))],
        out_specs=pl.BlockSpec((R2, N), lambda i: (0, 0)),
        out_shape=jax.ShapeDtypeStruct((R2, N), F32))(P, M)


def _pallas_dot(a, b):
    m, k = a.shape
    _, n = b.shape

    def kfn(a_ref, b_ref, o_ref):
        o_ref[...] = jnp.dot(a_ref[...], b_ref[...], preferred_element_type=F32)

    rb = 512
    if m * n * 4 > 1024 * 1024 and m % rb == 0:
        return pl.pallas_call(
            kfn,
            grid=(m // rb,),
            in_specs=[pl.BlockSpec((rb, k), lambda i: (i, 0)),
                      pl.BlockSpec((k, n), lambda i: (0, 0))],
            out_specs=pl.BlockSpec((rb, n), lambda i: (i, 0)),
            out_shape=jax.ShapeDtypeStruct((m, n), F32))(a, b)
    return pl.pallas_call(
        kfn, out_shape=jax.ShapeDtypeStruct((m, n), F32))(a, b)


def _bn6(z5, g, b):
    z6 = jax.lax.optimization_barrier(z5.transpose(4, 0, 1, 2, 3)[None])
    m = z6.mean(axis=(0, 2, 3, 4, 5), keepdims=True)
    v = z6.var(axis=(0, 2, 3, 4, 5), keepdims=True)
    y = (z6 - m) / jnp.sqrt(v + 1e-5) * g.reshape(1, -1, 1, 1, 1, 1) + b.reshape(1, -1, 1, 1, 1, 1)
    y = jax.lax.optimization_barrier(jax.nn.relu(y))
    return y[0].transpose(1, 2, 3, 4, 0)


def _conv_same(x5, M, g, b, Co):
    T, D, H, W, Ci = x5.shape
    xl = x5.reshape(T, D, H, W * Ci)
    xp = jnp.pad(xl, ((1, 1), (1, 1), (1, 1), (0, 0))).astype(BF16)
    xs = jnp.stack([xp[:, :, c:c + H] for c in range(3)])
    acc = _pallas_conv_same(xs, M.astype(BF16), T, D, H, W, Ci, Co)
    return _bn6(acc.reshape(T, D, H, W, Co), g, b)


def _conv_center(x5, w, g, b):
    Co, Ci = w.shape[:2]
    xr = jnp.zeros((8, Ci), BF16).at[0].set(x5.reshape(Ci).astype(BF16))
    acc = _pallas_dot(xr, w[:, :, 1, 1, 1, 1].T.astype(BF16))
    return _bn6(acc[0].reshape(1, 1, 1, 1, Co), g, b)


def _conv_down(x5, w, g, b):
    T, D, H, W, Ci = x5.shape
    Co = w.shape[0]
    T2, D2, H2, W2 = T // 2, D // 2, H // 2, W // 2
    xl = x5.reshape(T, D, H, W * Ci)
    P = jnp.stack([xl[a::2, bb::2, c::2].reshape(T2 * D2 * H2, W * Ci)
                   for a, bb, c in itertools.product(range(2), repeat=3)]).astype(BF16)
    M = _down_mats(w, W).astype(BF16)
    acc = _pallas_conv_down(P, M, T2 * D2 * H2, W, Ci, Co)
    return _bn6(acc.reshape(T2, D2, H2, W2, Co), g, b)


def _conv_up(x5, w, g, b):
    T, D, H, W, Ci = x5.shape
    Co = w.shape[0]
    V = T * D * H * W
    xr = x5.reshape(V, Ci).astype(BF16)
    if V < 8:
        xr = jnp.zeros((8, Ci), BF16).at[:V].set(xr)
    wu = jnp.transpose(w, (1, 2, 3, 4, 5, 0)).reshape(Ci, 16 * Co).astype(BF16)
    acc = _pallas_dot(xr, wu)[:V]
    z = acc.reshape(T, D, H, W, 2, 2, 2, 2, Co)
    z = z.transpose(0, 4, 1, 5, 2, 6, 3, 7, 8).reshape(2 * T, 2 * D, 2 * H, 2 * W, Co)
    return _bn6(z, g, b)


def _occ_prune(x5, wocc):
    T, D, H, W, C = x5.shape
    xl = x5.reshape(T * D * H, W * C).astype(BF16)
    Q = _occ_mat(wocc, W, C).astype(BF16)
    occ = _pallas_dot(xl, Q)
    occ5 = occ.reshape(T, D, H, W)
    keep = (jax.nn.sigmoid(occ5) >= 0.5).astype(F32)[..., None]
    pruned = jax.lax.optimization_barrier(x5 * keep)
    return occ5, pruned


def kernel(x, p):
    _, Cin, T, D, H, W = x.shape
    x5 = x[0].transpose(1, 2, 3, 4, 0)

    stem = _conv_same(x5, _blockdiag_mats(p['stem_w'], W), p['stem_g'], p['stem_b'], 16)

    def enc(h, i, Co):
        h = _conv_down(h, p['enc%d_dw' % i], p['enc%d_dg' % i], p['enc%d_db' % i])
        S = h.shape[0]
        if S == 1:
            h = _conv_center(h, p['enc%d_rw' % i], p['enc%d_rg' % i], p['enc%d_rb' % i])
        else:
            h = _conv_same(h, _blockdiag_mats(p['enc%d_rw' % i], S),
                           p['enc%d_rg' % i], p['enc%d_rb' % i], Co)
        return h

    e1 = enc(stem, 1, 16)
    e2 = enc(e1, 2, 32)
    e3 = enc(e2, 3, 64)
    lat = enc(e3, 4, 128)

    def dec(h, skip, n, Co):
        h = _conv_up(h, p[n + '_uw'], p[n + '_ug'], p[n + '_ub'])
        h = jnp.concatenate([h, skip], axis=-1)
        S = h.shape[0]
        h = _conv_same(h, _blockdiag_mats(p[n + '_fw'], S), p[n + '_fg'], p[n + '_fb'], Co)
        return h

    d3 = dec(lat, e3, 'dec3', 64)
    occ3, d3 = _occ_prune(d3, p['occ3_w'])
    d2 = dec(d3, e2, 'dec2', 32)
    occ2, d2 = _occ_prune(d2, p['occ2_w'])
    d1 = dec(d2, e1, 'dec1', 16)
    occ1, d1 = _occ_prune(d1, p['occ1_w'])
    d0 = dec(d1, stem, 'dec0', 16)
    occ0, d0 = _occ_prune(d0, p['occ0_w'])

    Q3 = _occ_mat(p['off_w'], W, 16)
    off = _pallas_dot(d0.reshape(T * D * H, W * 16).astype(BF16), Q3.astype(BF16))
    offsets = jax.nn.sigmoid(off.reshape(T, D, H, W, 3)).transpose(4, 0, 1, 2, 3)[None]

    def to_out(o):
        return o[None, None]

    return offsets, to_out(occ3), to_out(occ2), to_out(occ1), to_out(occ0)


# speedup vs baseline: 3.7040x; 1.2406x over previous
import itertools
import jax
import jax.numpy as jnp
from jax.experimental import pallas as pl

F32 = jnp.float32
BF16 = jnp.bfloat16


def _blockdiag_mats(w, W):
    Co, Ci = w.shape[:2]
    wi = jnp.arange(W)[:, None]
    wo = jnp.arange(W)[None, :]
    mats = []
    for a, b, c, d in itertools.product(range(3), repeat=4):
        sel = (wi - wo + 1 == d).astype(w.dtype)
        M = jnp.einsum('wv,io->wivo', sel, w[:, :, a, b, c, d].T).reshape(W * Ci, W * Co)
        mats.append(M)
    return jnp.stack(mats)


def _down_mats(w, W):
    Co, Ci = w.shape[:2]
    W2 = W // 2
    wi = jnp.arange(W)[:, None]
    wo = jnp.arange(W2)[None, :]
    mats = []
    for a, b, c, d in itertools.product(range(2), repeat=4):
        sel = (wi == 2 * wo + d).astype(w.dtype)
        M = jnp.einsum('wv,io->wivo', sel, w[:, :, a, b, c, d].T).reshape(W * Ci, W2 * Co)
        mats.append(M)
    return jnp.stack(mats)


def _occ_mat(wocc, W, C):
    Co = wocc.shape[0]
    q = wocc[:, :, 0, 0, 0, 0].T
    eye = jnp.eye(W, dtype=q.dtype)
    return jnp.einsum('wv,io->wivo', eye, q).reshape(W * C, W * Co)


def _pallas_conv_same(xs, M, T, D, H, W, Ci, Co):
    R = T * D * H
    RB = D * H
    L = W * Ci
    N = W * Co

    if RB % 8 == 0:
        G = min(T, max(1, 1024 // RB))
        TG = T // G
        GR = G * RB

        if L == 512 and N == 256:
            M0 = M[:, 0:384, 0:128]
            M1 = M[:, 128:512, 128:256]

            def kfn2(xs_ref, m0_ref, m1_ref, o_ref):
                t = pl.program_id(0)
                i = pl.program_id(1)
                a = i // 27
                b = (i // 9) % 3
                c = (i // 3) % 3
                patch = xs_ref[pl.ds(c, 1), pl.ds(t * G + a, G), pl.ds(b, D), :, :].reshape(GR, L)
                dd0 = jnp.dot(patch[:, 0:384], m0_ref[i], preferred_element_type=F32)
                dd1 = jnp.dot(patch[:, 128:512], m1_ref[i], preferred_element_type=F32)

                @pl.when(i == 0)
                def _():
                    o_ref[:, 0:128] = dd0
                    o_ref[:, 128:256] = dd1

                @pl.when(i > 0)
                def _():
                    o_ref[:, 0:128] = o_ref[:, 0:128] + dd0
                    o_ref[:, 128:256] = o_ref[:, 128:256] + dd1

            return pl.pallas_call(
                kfn2,
                grid=(TG, 81),
                in_specs=[pl.BlockSpec((3, T + 2, D + 2, H, L), lambda t, i: (0, 0, 0, 0, 0)),
                          pl.BlockSpec((81, 384, 128), lambda t, i: (0, 0, 0)),
                          pl.BlockSpec((81, 384, 128), lambda t, i: (0, 0, 0))],
                out_specs=pl.BlockSpec((GR, N), lambda t, i: (t, 0)),
                out_shape=jax.ShapeDtypeStruct((R, N), F32))(xs, M0, M1)

        def kfn(xs_ref, m_ref, o_ref):
            t = pl.program_id(0)
            i = pl.program_id(1)
            a = i // 27
            b = (i // 9) % 3
            c = (i // 3) % 3
            patch = xs_ref[pl.ds(c, 1), pl.ds(t * G + a, G), pl.ds(b, D), :, :].reshape(GR, L)
            dd = jnp.dot(patch, m_ref[i], preferred_element_type=F32)

            @pl.when(i == 0)
            def _():
                o_ref[...] = dd

            @pl.when(i > 0)
            def _():
                o_ref[...] = o_ref[...] + dd

        return pl.pallas_call(
            kfn,
            grid=(TG, 81),
            in_specs=[pl.BlockSpec((3, T + 2, D + 2, H, L), lambda t, i: (0, 0, 0, 0, 0)),
                      pl.BlockSpec((81, L, N), lambda t, i: (0, 0, 0))],
            out_specs=pl.BlockSpec((GR, N), lambda t, i: (t, 0)),
            out_shape=jax.ShapeDtypeStruct((R, N), F32))(xs, M)

    def kfn_s(xs_ref, m_ref, o_ref):
        i = pl.program_id(0)
        a = i // 27
        b = (i // 9) % 3
        c = (i // 3) % 3
        patch = xs_ref[pl.ds(c, 1), pl.ds(a, T), pl.ds(b, D), :, :].reshape(R, L)
        dd = jnp.dot(patch, m_ref[i], preferred_element_type=F32)

        @pl.when(i == 0)
        def _():
            o_ref[...] = dd

        @pl.when(i > 0)
        def _():
            o_ref[...] = o_ref[...] + dd

    return pl.pallas_call(
        kfn_s,
        grid=(81,),
        in_specs=[pl.BlockSpec((3, T + 2, D + 2, H, L), lambda i: (0, 0, 0, 0, 0)),
                  pl.BlockSpec((81, L, N), lambda i: (0, 0, 0))],
        out_specs=pl.BlockSpec((R, N), lambda i: (0, 0)),
        out_shape=jax.ShapeDtypeStruct((R, N), F32))(xs, M)


def _pallas_conv_down(P, M, R2, W, Ci, Co):
    W2 = W // 2
    L = W * Ci
    N = W2 * Co

    def kfn(p_ref, m_ref, o_ref):
        i = pl.program_id(0)
        patch = p_ref[i // 2]
        dd = jnp.dot(patch, m_ref[i], preferred_element_type=F32)

        @pl.when(i == 0)
        def _():
            o_ref[...] = dd

        @pl.when(i > 0)
        def _():
            o_ref[...] = o_ref[...] + dd

    return pl.pallas_call(
        kfn,
        grid=(16,),
        in_specs=[pl.BlockSpec((8, R2, L), lambda i: (0, 0, 0)),
                  pl.BlockSpec((16, L, N), lambda i: (0, 0, 0))],
        out_specs=pl.BlockSpec((R2, N), lambda i: (0, 0)),
        out_shape=jax.ShapeDtypeStruct((R2, N), F32))(P, M)


def _pallas_dot(a, b):
    m, k = a.shape
    _, n = b.shape

    def kfn(a_ref, b_ref, o_ref):
        o_ref[...] = jnp.dot(a_ref[...], b_ref[...], preferred_element_type=F32)

    rb = 512
    if m * n * 4 > 1024 * 1024 and m % rb == 0:
        return pl.pallas_call(
            kfn,
            grid=(m // rb,),
            in_specs=[pl.BlockSpec((rb, k), lambda i: (i, 0)),
                      pl.BlockSpec((k, n), lambda i: (0, 0))],
            out_specs=pl.BlockSpec((rb, n), lambda i: (i, 0)),
            out_shape=jax.ShapeDtypeStruct((m, n), F32))(a, b)
    return pl.pallas_call(
        kfn, out_shape=jax.ShapeDtypeStruct((m, n), F32))(a, b)


def _bn6(z5, g, b):
    z6 = jax.lax.optimization_barrier(z5.transpose(4, 0, 1, 2, 3)[None])
    m = z6.mean(axis=(0, 2, 3, 4, 5), keepdims=True)
    v = z6.var(axis=(0, 2, 3, 4, 5), keepdims=True)
    y = (z6 - m) / jnp.sqrt(v + 1e-5) * g.reshape(1, -1, 1, 1, 1, 1) + b.reshape(1, -1, 1, 1, 1, 1)
    y = jax.lax.optimization_barrier(jax.nn.relu(y))
    return y[0].transpose(1, 2, 3, 4, 0)


def _conv_same(x5, M, g, b, Co):
    T, D, H, W, Ci = x5.shape
    xl = x5.reshape(T, D, H, W * Ci)
    xp = jnp.pad(xl, ((1, 1), (1, 1), (1, 1), (0, 0))).astype(BF16)
    xs = jnp.stack([xp[:, :, c:c + H] for c in range(3)])
    acc = _pallas_conv_same(xs, M.astype(BF16), T, D, H, W, Ci, Co)
    return _bn6(acc.reshape(T, D, H, W, Co), g, b)


def _conv_center(x5, w, g, b):
    Co, Ci = w.shape[:2]
    xr = jnp.zeros((8, Ci), BF16).at[0].set(x5.reshape(Ci).astype(BF16))
    acc = _pallas_dot(xr, w[:, :, 1, 1, 1, 1].T.astype(BF16))
    return _bn6(acc[0].reshape(1, 1, 1, 1, Co), g, b)


def _conv_down(x5, w, g, b):
    T, D, H, W, Ci = x5.shape
    Co = w.shape[0]
    T2, D2, H2, W2 = T // 2, D // 2, H // 2, W // 2
    xl = x5.reshape(T, D, H, W * Ci)
    P = jnp.stack([xl[a::2, bb::2, c::2].reshape(T2 * D2 * H2, W * Ci)
                   for a, bb, c in itertools.product(range(2), repeat=3)]).astype(BF16)
    M = _down_mats(w, W).astype(BF16)
    acc = _pallas_conv_down(P, M, T2 * D2 * H2, W, Ci, Co)
    return _bn6(acc.reshape(T2, D2, H2, W2, Co), g, b)


def _conv_up(x5, w, g, b):
    T, D, H, W, Ci = x5.shape
    Co = w.shape[0]
    V = T * D * H * W
    xr = x5.reshape(V, Ci).astype(BF16)
    if V < 8:
        xr = jnp.zeros((8, Ci), BF16).at[:V].set(xr)
    wu = jnp.transpose(w, (1, 2, 3, 4, 5, 0)).reshape(Ci, 16 * Co).astype(BF16)
    acc = _pallas_dot(xr, wu)[:V]
    z = acc.reshape(T, D, H, W, 2, 2, 2, 2, Co)
    z = z.transpose(0, 4, 1, 5, 2, 6, 3, 7, 8).reshape(2 * T, 2 * D, 2 * H, 2 * W, Co)
    return _bn6(z, g, b)


def _occ_prune(x5, wocc):
    T, D, H, W, C = x5.shape
    xl = x5.reshape(T * D * H, W * C).astype(BF16)
    Q = _occ_mat(wocc, W, C).astype(BF16)
    occ = _pallas_dot(xl, Q)
    occ5 = occ.reshape(T, D, H, W)
    keep = (jax.nn.sigmoid(occ5) >= 0.5).astype(F32)[..., None]
    pruned = jax.lax.optimization_barrier(x5 * keep)
    return occ5, pruned


def kernel(x, p):
    _, Cin, T, D, H, W = x.shape
    x5 = x[0].transpose(1, 2, 3, 4, 0)

    stem = _conv_same(x5, _blockdiag_mats(p['stem_w'], W), p['stem_g'], p['stem_b'], 16)

    def enc(h, i, Co):
        h = _conv_down(h, p['enc%d_dw' % i], p['enc%d_dg' % i], p['enc%d_db' % i])
        S = h.shape[0]
        if S == 1:
            h = _conv_center(h, p['enc%d_rw' % i], p['enc%d_rg' % i], p['enc%d_rb' % i])
        else:
            h = _conv_same(h, _blockdiag_mats(p['enc%d_rw' % i], S),
                           p['enc%d_rg' % i], p['enc%d_rb' % i], Co)
        return h

    e1 = enc(stem, 1, 16)
    e2 = enc(e1, 2, 32)
    e3 = enc(e2, 3, 64)
    lat = enc(e3, 4, 128)

    def dec(h, skip, n, Co):
        h = _conv_up(h, p[n + '_uw'], p[n + '_ug'], p[n + '_ub'])
        h = jnp.concatenate([h, skip], axis=-1)
        S = h.shape[0]
        h = _conv_same(h, _blockdiag_mats(p[n + '_fw'], S), p[n + '_fg'], p[n + '_fb'], Co)
        return h

    d3 = dec(lat, e3, 'dec3', 64)
    occ3, d3 = _occ_prune(d3, p['occ3_w'])
    d2 = dec(d3, e2, 'dec2', 32)
    occ2, d2 = _occ_prune(d2, p['occ2_w'])
    d1 = dec(d2, e1, 'dec1', 16)
    occ1, d1 = _occ_prune(d1, p['occ1_w'])
    d0 = dec(d1, stem, 'dec0', 16)
    occ0, d0 = _occ_prune(d0, p['occ0_w'])

    Q3 = _occ_mat(p['off_w'], W, 16)
    off = _pallas_dot(d0.reshape(T * D * H, W * 16).astype(BF16), Q3.astype(BF16))
    offsets = jax.nn.sigmoid(off.reshape(T, D, H, W, 3)).transpose(4, 0, 1, 2, 3)[None]

    def to_out(o):
        return o[None, None]

    return offsets, to_out(occ3), to_out(occ2), to_out(occ1), to_out(occ0)


# speedup vs baseline: 3.8103x; 1.0287x over previous
import itertools
import jax
import jax.numpy as jnp
from jax.experimental import pallas as pl

F32 = jnp.float32
BF16 = jnp.bfloat16


def _blockdiag_mats(w, W):
    Co, Ci = w.shape[:2]
    wi = jnp.arange(W)[:, None]
    wo = jnp.arange(W)[None, :]
    mats = []
    for a, b, c, d in itertools.product(range(3), repeat=4):
        sel = (wi - wo + 1 == d).astype(w.dtype)
        M = jnp.einsum('wv,io->wivo', sel, w[:, :, a, b, c, d].T).reshape(W * Ci, W * Co)
        mats.append(M)
    return jnp.stack(mats)


def _down_mats(w, W):
    Co, Ci = w.shape[:2]
    W2 = W // 2
    wi = jnp.arange(W)[:, None]
    wo = jnp.arange(W2)[None, :]
    mats = []
    for a, b, c, d in itertools.product(range(2), repeat=4):
        sel = (wi == 2 * wo + d).astype(w.dtype)
        M = jnp.einsum('wv,io->wivo', sel, w[:, :, a, b, c, d].T).reshape(W * Ci, W2 * Co)
        mats.append(M)
    return jnp.stack(mats)


def _occ_mat(wocc, W, C):
    Co = wocc.shape[0]
    q = wocc[:, :, 0, 0, 0, 0].T
    eye = jnp.eye(W, dtype=q.dtype)
    return jnp.einsum('wv,io->wivo', eye, q).reshape(W * C, W * Co)


def _pallas_conv_same(xs, M, T, D, H, W, Ci, Co):
    R = T * D * H
    RB = D * H
    L = W * Ci
    N = W * Co

    if RB % 8 == 0:
        G = min(T, max(1, 2048 // RB))
        TG = T // G
        GR = G * RB

        if L == 512 and N == 256:
            M0 = M[:, 0:384, 0:128]
            M1 = M[:, 128:512, 128:256]

            def kfn2(xs_ref, m0_ref, m1_ref, o_ref):
                t = pl.program_id(0)
                i = pl.program_id(1)
                a = i // 27
                b = (i // 9) % 3
                c = (i // 3) % 3
                patch = xs_ref[pl.ds(c, 1), pl.ds(t * G + a, G), pl.ds(b, D), :, :].reshape(GR, L)
                dd0 = jnp.dot(patch[:, 0:384], m0_ref[i], preferred_element_type=F32)
                dd1 = jnp.dot(patch[:, 128:512], m1_ref[i], preferred_element_type=F32)

                @pl.when(i == 0)
                def _():
                    o_ref[:, 0:128] = dd0
                    o_ref[:, 128:256] = dd1

                @pl.when(i > 0)
                def _():
                    o_ref[:, 0:128] = o_ref[:, 0:128] + dd0
                    o_ref[:, 128:256] = o_ref[:, 128:256] + dd1

            return pl.pallas_call(
                kfn2,
                grid=(TG, 81),
                in_specs=[pl.BlockSpec((3, T + 2, D + 2, H, L), lambda t, i: (0, 0, 0, 0, 0)),
                          pl.BlockSpec((81, 384, 128), lambda t, i: (0, 0, 0)),
                          pl.BlockSpec((81, 384, 128), lambda t, i: (0, 0, 0))],
                out_specs=pl.BlockSpec((GR, N), lambda t, i: (t, 0)),
                out_shape=jax.ShapeDtypeStruct((R, N), F32))(xs, M0, M1)

        def kfn(xs_ref, m_ref, o_ref):
            t = pl.program_id(0)
            i = pl.program_id(1)
            a = i // 27
            b = (i // 9) % 3
            c = (i // 3) % 3
            patch = xs_ref[pl.ds(c, 1), pl.ds(t * G + a, G), pl.ds(b, D), :, :].reshape(GR, L)
            dd = jnp.dot(patch, m_ref[i], preferred_element_type=F32)

            @pl.when(i == 0)
            def _():
                o_ref[...] = dd

            @pl.when(i > 0)
            def _():
                o_ref[...] = o_ref[...] + dd

        return pl.pallas_call(
            kfn,
            grid=(TG, 81),
            in_specs=[pl.BlockSpec((3, T + 2, D + 2, H, L), lambda t, i: (0, 0, 0, 0, 0)),
                      pl.BlockSpec((81, L, N), lambda t, i: (0, 0, 0))],
            out_specs=pl.BlockSpec((GR, N), lambda t, i: (t, 0)),
            out_shape=jax.ShapeDtypeStruct((R, N), F32))(xs, M)

    def kfn_s(xs_ref, m_ref, o_ref):
        i = pl.program_id(0)
        a = i // 27
        b = (i // 9) % 3
        c = (i // 3) % 3
        patch = xs_ref[pl.ds(c, 1), pl.ds(a, T), pl.ds(b, D), :, :].reshape(R, L)
        dd = jnp.dot(patch, m_ref[i], preferred_element_type=F32)

        @pl.when(i == 0)
        def _():
            o_ref[...] = dd

        @pl.when(i > 0)
        def _():
            o_ref[...] = o_ref[...] + dd

    return pl.pallas_call(
        kfn_s,
        grid=(81,),
        in_specs=[pl.BlockSpec((3, T + 2, D + 2, H, L), lambda i: (0, 0, 0, 0, 0)),
                  pl.BlockSpec((81, L, N), lambda i: (0, 0, 0))],
        out_specs=pl.BlockSpec((R, N), lambda i: (0, 0)),
        out_shape=jax.ShapeDtypeStruct((R, N), F32))(xs, M)


def _pallas_conv_down(P, M, R2, W, Ci, Co):
    W2 = W // 2
    L = W * Ci
    N = W2 * Co

    def kfn(p_ref, m_ref, o_ref):
        i = pl.program_id(0)
        patch = p_ref[i // 2]
        dd = jnp.dot(patch, m_ref[i], preferred_element_type=F32)

        @pl.when(i == 0)
        def _():
            o_ref[...] = dd

        @pl.when(i > 0)
        def _():
            o_ref[...] = o_ref[...] + dd

    return pl.pallas_call(
        kfn,
        grid=(16,),
        in_specs=[pl.BlockSpec((8, R2, L), lambda i: (0, 0, 0)),
                  pl.BlockSpec((16, L, N), lambda i: (0, 0, 0))],
        out_specs=pl.BlockSpec((R2, N), lambda i: (0, 0)),
        out_shape=jax.ShapeDtypeStruct((R2, N), F32))(P, M)


def _pallas_dot(a, b):
    m, k = a.shape
    _, n = b.shape

    def kfn(a_ref, b_ref, o_ref):
        o_ref[...] = jnp.dot(a_ref[...], b_ref[...], preferred_element_type=F32)

    rb = 512
    if m * n * 4 > 1024 * 1024 and m % rb == 0:
        return pl.pallas_call(
            kfn,
            grid=(m // rb,),
            in_specs=[pl.BlockSpec((rb, k), lambda i: (i, 0)),
                      pl.BlockSpec((k, n), lambda i: (0, 0))],
            out_specs=pl.BlockSpec((rb, n), lambda i: (i, 0)),
            out_shape=jax.ShapeDtypeStruct((m, n), F32))(a, b)
    return pl.pallas_call(
        kfn, out_shape=jax.ShapeDtypeStruct((m, n), F32))(a, b)


def _bn6(z5, g, b):
    z6 = jax.lax.optimization_barrier(z5.transpose(4, 0, 1, 2, 3)[None])
    m = z6.mean(axis=(0, 2, 3, 4, 5), keepdims=True)
    v = z6.var(axis=(0, 2, 3, 4, 5), keepdims=True)
    y = (z6 - m) / jnp.sqrt(v + 1e-5) * g.reshape(1, -1, 1, 1, 1, 1) + b.reshape(1, -1, 1, 1, 1, 1)
    y = jax.lax.optimization_barrier(jax.nn.relu(y))
    return y[0].transpose(1, 2, 3, 4, 0)


def _conv_same(x5, M, g, b, Co):
    T, D, H, W, Ci = x5.shape
    xl = x5.reshape(T, D, H, W * Ci)
    xp = jnp.pad(xl, ((1, 1), (1, 1), (1, 1), (0, 0))).astype(BF16)
    xs = jnp.stack([xp[:, :, c:c + H] for c in range(3)])
    acc = _pallas_conv_same(xs, M.astype(BF16), T, D, H, W, Ci, Co)
    return _bn6(acc.reshape(T, D, H, W, Co), g, b)


def _conv_center(x5, w, g, b):
    Co, Ci = w.shape[:2]
    xr = jnp.zeros((8, Ci), BF16).at[0].set(x5.reshape(Ci).astype(BF16))
    acc = _pallas_dot(xr, w[:, :, 1, 1, 1, 1].T.astype(BF16))
    return _bn6(acc[0].reshape(1, 1, 1, 1, Co), g, b)


def _conv_down(x5, w, g, b):
    T, D, H, W, Ci = x5.shape
    Co = w.shape[0]
    T2, D2, H2, W2 = T // 2, D // 2, H // 2, W // 2
    xl = x5.reshape(T, D, H, W * Ci)
    P = jnp.stack([xl[a::2, bb::2, c::2].reshape(T2 * D2 * H2, W * Ci)
                   for a, bb, c in itertools.product(range(2), repeat=3)]).astype(BF16)
    M = _down_mats(w, W).astype(BF16)
    acc = _pallas_conv_down(P, M, T2 * D2 * H2, W, Ci, Co)
    return _bn6(acc.reshape(T2, D2, H2, W2, Co), g, b)


def _conv_up(x5, w, g, b):
    T, D, H, W, Ci = x5.shape
    Co = w.shape[0]
    V = T * D * H * W
    xr = x5.reshape(V, Ci).astype(BF16)
    if V < 8:
        xr = jnp.zeros((8, Ci), BF16).at[:V].set(xr)
    wu = jnp.transpose(w, (1, 2, 3, 4, 5, 0)).reshape(Ci, 16 * Co).astype(BF16)
    acc = _pallas_dot(xr, wu)[:V]
    z = acc.reshape(T, D, H, W, 2, 2, 2, 2, Co)
    z = z.transpose(0, 4, 1, 5, 2, 6, 3, 7, 8).reshape(2 * T, 2 * D, 2 * H, 2 * W, Co)
    return _bn6(z, g, b)


def _occ_prune(x5, wocc):
    T, D, H, W, C = x5.shape
    xl = x5.reshape(T * D * H, W * C).astype(BF16)
    Q = _occ_mat(wocc, W, C).astype(BF16)
    occ = _pallas_dot(xl, Q)
    occ5 = occ.reshape(T, D, H, W)
    keep = (jax.nn.sigmoid(occ5) >= 0.5).astype(F32)[..., None]
    pruned = jax.lax.optimization_barrier(x5 * keep)
    return occ5, pruned


def kernel(x, p):
    _, Cin, T, D, H, W = x.shape
    x5 = x[0].transpose(1, 2, 3, 4, 0)

    stem = _conv_same(x5, _blockdiag_mats(p['stem_w'], W), p['stem_g'], p['stem_b'], 16)

    def enc(h, i, Co):
        h = _conv_down(h, p['enc%d_dw' % i], p['enc%d_dg' % i], p['enc%d_db' % i])
        S = h.shape[0]
        if S == 1:
            h = _conv_center(h, p['enc%d_rw' % i], p['enc%d_rg' % i], p['enc%d_rb' % i])
        else:
            h = _conv_same(h, _blockdiag_mats(p['enc%d_rw' % i], S),
                           p['enc%d_rg' % i], p['enc%d_rb' % i], Co)
        return h

    e1 = enc(stem, 1, 16)
    e2 = enc(e1, 2, 32)
    e3 = enc(e2, 3, 64)
    lat = enc(e3, 4, 128)

    def dec(h, skip, n, Co):
        h = _conv_up(h, p[n + '_uw'], p[n + '_ug'], p[n + '_ub'])
        h = jnp.concatenate([h, skip], axis=-1)
        S = h.shape[0]
        h = _conv_same(h, _blockdiag_mats(p[n + '_fw'], S), p[n + '_fg'], p[n + '_fb'], Co)
        return h

    d3 = dec(lat, e3, 'dec3', 64)
    occ3, d3 = _occ_prune(d3, p['occ3_w'])
    d2 = dec(d3, e2, 'dec2', 32)
    occ2, d2 = _occ_prune(d2, p['occ2_w'])
    d1 = dec(d2, e1, 'dec1', 16)
    occ1, d1 = _occ_prune(d1, p['occ1_w'])
    d0 = dec(d1, stem, 'dec0', 16)
    occ0, d0 = _occ_prune(d0, p['occ0_w'])

    Q3 = _occ_mat(p['off_w'], W, 16)
    off = _pallas_dot(d0.reshape(T * D * H, W * 16).astype(BF16), Q3.astype(BF16))
    offsets = jax.nn.sigmoid(off.reshape(T, D, H, W, 3)).transpose(4, 0, 1, 2, 3)[None]

    def to_out(o):
        return o[None, None]

    return offsets, to_out(occ3), to_out(occ2), to_out(occ1), to_out(occ0)


# speedup vs baseline: 3.8555x; 1.0119x over previous
import itertools
import jax
import jax.numpy as jnp
from jax.experimental import pallas as pl

F32 = jnp.float32
BF16 = jnp.bfloat16


def _blockdiag_mats(w, W):
    Co, Ci = w.shape[:2]
    wi = jnp.arange(W)[:, None]
    wo = jnp.arange(W)[None, :]
    mats = []
    for a, b, c, d in itertools.product(range(3), repeat=4):
        sel = (wi - wo + 1 == d).astype(w.dtype)
        M = jnp.einsum('wv,io->wivo', sel, w[:, :, a, b, c, d].T).reshape(W * Ci, W * Co)
        mats.append(M)
    return jnp.stack(mats)


def _down_mats(w, W):
    Co, Ci = w.shape[:2]
    W2 = W // 2
    wi = jnp.arange(W)[:, None]
    wo = jnp.arange(W2)[None, :]
    mats = []
    for a, b, c, d in itertools.product(range(2), repeat=4):
        sel = (wi == 2 * wo + d).astype(w.dtype)
        M = jnp.einsum('wv,io->wivo', sel, w[:, :, a, b, c, d].T).reshape(W * Ci, W2 * Co)
        mats.append(M)
    return jnp.stack(mats)


def _occ_mat(wocc, W, C):
    Co = wocc.shape[0]
    q = wocc[:, :, 0, 0, 0, 0].T
    eye = jnp.eye(W, dtype=q.dtype)
    return jnp.einsum('wv,io->wivo', eye, q).reshape(W * C, W * Co)


def _pallas_conv_same(xs, M, T, D, H, W, Ci, Co):
    R = T * D * H
    RB = D * H
    L = W * Ci
    N = W * Co

    if RB % 8 == 0:
        G = min(T, max(1, 2048 // RB))
        TG = T // G
        GR = G * RB

        if L == 512 and N == 256:
            M0 = M[:, 0:384, 0:128]
            M1 = M[:, 128:512, 128:256]

            def kfn2(xs_ref, m0_ref, m1_ref, o_ref):
                t = pl.program_id(0)
                i = pl.program_id(1)
                a = i // 27
                b = (i // 9) % 3
                c = (i // 3) % 3
                patch = xs_ref[pl.ds(c, 1), pl.ds(t * G + a, G), pl.ds(b, D), :, :].reshape(GR, L)
                dd0 = jnp.dot(patch[:, 0:384], m0_ref[i], preferred_element_type=F32)
                dd1 = jnp.dot(patch[:, 128:512], m1_ref[i], preferred_element_type=F32)

                @pl.when(i == 0)
                def _():
                    o_ref[:, 0:128] = dd0
                    o_ref[:, 128:256] = dd1

                @pl.when(i > 0)
                def _():
                    o_ref[:, 0:128] = o_ref[:, 0:128] + dd0
                    o_ref[:, 128:256] = o_ref[:, 128:256] + dd1

            return pl.pallas_call(
                kfn2,
                grid=(TG, 81),
                in_specs=[pl.BlockSpec((3, T + 2, D + 2, H, L), lambda t, i: (0, 0, 0, 0, 0)),
                          pl.BlockSpec((81, 384, 128), lambda t, i: (0, 0, 0)),
                          pl.BlockSpec((81, 384, 128), lambda t, i: (0, 0, 0))],
                out_specs=pl.BlockSpec((GR, N), lambda t, i: (t, 0)),
                out_shape=jax.ShapeDtypeStruct((R, N), F32))(xs, M0, M1)

        def kfn(xs_ref, m_ref, o_ref):
            t = pl.program_id(0)
            i = pl.program_id(1)
            a = i // 27
            b = (i // 9) % 3
            c = (i // 3) % 3
            patch = xs_ref[pl.ds(c, 1), pl.ds(t * G + a, G), pl.ds(b, D), :, :].reshape(GR, L)
            dd = jnp.dot(patch, m_ref[i], preferred_element_type=F32)

            @pl.when(i == 0)
            def _():
                o_ref[...] = dd

            @pl.when(i > 0)
            def _():
                o_ref[...] = o_ref[...] + dd

        return pl.pallas_call(
            kfn,
            grid=(TG, 81),
            in_specs=[pl.BlockSpec((3, T + 2, D + 2, H, L), lambda t, i: (0, 0, 0, 0, 0)),
                      pl.BlockSpec((81, L, N), lambda t, i: (0, 0, 0))],
            out_specs=pl.BlockSpec((GR, N), lambda t, i: (t, 0)),
            out_shape=jax.ShapeDtypeStruct((R, N), F32))(xs, M)

    def kfn_s(xs_ref, m_ref, o_ref):
        i = pl.program_id(0)
        a = i // 27
        b = (i // 9) % 3
        c = (i // 3) % 3
        patch = xs_ref[pl.ds(c, 1), pl.ds(a, T), pl.ds(b, D), :, :].reshape(R, L)
        dd = jnp.dot(patch, m_ref[i], preferred_element_type=F32)

        @pl.when(i == 0)
        def _():
            o_ref[...] = dd

        @pl.when(i > 0)
        def _():
            o_ref[...] = o_ref[...] + dd

    return pl.pallas_call(
        kfn_s,
        grid=(81,),
        in_specs=[pl.BlockSpec((3, T + 2, D + 2, H, L), lambda i: (0, 0, 0, 0, 0)),
                  pl.BlockSpec((81, L, N), lambda i: (0, 0, 0))],
        out_specs=pl.BlockSpec((R, N), lambda i: (0, 0)),
        out_shape=jax.ShapeDtypeStruct((R, N), F32))(xs, M)


def _pallas_conv_down(P, M, R2, W, Ci, Co):
    W2 = W // 2
    L = W * Ci
    N = W2 * Co

    def kfn(p_ref, m_ref, o_ref):
        i = pl.program_id(0)
        patch = p_ref[i // 2]
        dd = jnp.dot(patch, m_ref[i], preferred_element_type=F32)

        @pl.when(i == 0)
        def _():
            o_ref[...] = dd

        @pl.when(i > 0)
        def _():
            o_ref[...] = o_ref[...] + dd

    return pl.pallas_call(
        kfn,
        grid=(16,),
        in_specs=[pl.BlockSpec((8, R2, L), lambda i: (0, 0, 0)),
                  pl.BlockSpec((16, L, N), lambda i: (0, 0, 0))],
        out_specs=pl.BlockSpec((R2, N), lambda i: (0, 0)),
        out_shape=jax.ShapeDtypeStruct((R2, N), F32))(P, M)


def _pallas_dot(a, b):
    m, k = a.shape
    _, n = b.shape

    def kfn(a_ref, b_ref, o_ref):
        o_ref[...] = jnp.dot(a_ref[...], b_ref[...], preferred_element_type=F32)

    rb = 512
    if m * n * 4 > 1024 * 1024 and m % rb == 0:
        return pl.pallas_call(
            kfn,
            grid=(m // rb,),
            in_specs=[pl.BlockSpec((rb, k), lambda i: (i, 0)),
                      pl.BlockSpec((k, n), lambda i: (0, 0))],
            out_specs=pl.BlockSpec((rb, n), lambda i: (i, 0)),
            out_shape=jax.ShapeDtypeStruct((m, n), F32))(a, b)
    return pl.pallas_call(
        kfn, out_shape=jax.ShapeDtypeStruct((m, n), F32))(a, b)


def _bn6(z5, g, b):
    z6 = jax.lax.optimization_barrier(z5.transpose(4, 0, 1, 2, 3)[None])
    m = z6.mean(axis=(0, 2, 3, 4, 5), keepdims=True)
    v = z6.var(axis=(0, 2, 3, 4, 5), keepdims=True)
    y = (z6 - m) / jnp.sqrt(v + 1e-5) * g.reshape(1, -1, 1, 1, 1, 1) + b.reshape(1, -1, 1, 1, 1, 1)
    y = jax.lax.optimization_barrier(jax.nn.relu(y).astype(BF16))
    return y[0].transpose(1, 2, 3, 4, 0)


def _conv_same(x5, M, g, b, Co):
    T, D, H, W, Ci = x5.shape
    xl = x5.reshape(T, D, H, W * Ci)
    xp = jnp.pad(xl, ((1, 1), (1, 1), (1, 1), (0, 0))).astype(BF16)
    xs = jnp.stack([xp[:, :, c:c + H] for c in range(3)])
    acc = _pallas_conv_same(xs, M.astype(BF16), T, D, H, W, Ci, Co)
    return _bn6(acc.reshape(T, D, H, W, Co), g, b)


def _conv_center(x5, w, g, b):
    Co, Ci = w.shape[:2]
    xr = jnp.zeros((8, Ci), BF16).at[0].set(x5.reshape(Ci).astype(BF16))
    acc = _pallas_dot(xr, w[:, :, 1, 1, 1, 1].T.astype(BF16))
    return _bn6(acc[0].reshape(1, 1, 1, 1, Co), g, b)


def _conv_down(x5, w, g, b):
    T, D, H, W, Ci = x5.shape
    Co = w.shape[0]
    T2, D2, H2, W2 = T // 2, D // 2, H // 2, W // 2
    xl = x5.reshape(T, D, H, W * Ci)
    P = jnp.stack([xl[a::2, bb::2, c::2].reshape(T2 * D2 * H2, W * Ci)
                   for a, bb, c in itertools.product(range(2), repeat=3)]).astype(BF16)
    M = _down_mats(w, W).astype(BF16)
    acc = _pallas_conv_down(P, M, T2 * D2 * H2, W, Ci, Co)
    return _bn6(acc.reshape(T2, D2, H2, W2, Co), g, b)


def _conv_up(x5, w, g, b):
    T, D, H, W, Ci = x5.shape
    Co = w.shape[0]
    V = T * D * H * W
    xr = x5.reshape(V, Ci).astype(BF16)
    if V < 8:
        xr = jnp.zeros((8, Ci), BF16).at[:V].set(xr)
    wu = jnp.transpose(w, (1, 2, 3, 4, 5, 0)).reshape(Ci, 16 * Co).astype(BF16)
    acc = _pallas_dot(xr, wu)[:V]
    z = acc.reshape(T, D, H, W, 2, 2, 2, 2, Co)
    z = z.transpose(0, 4, 1, 5, 2, 6, 3, 7, 8).reshape(2 * T, 2 * D, 2 * H, 2 * W, Co)
    return _bn6(z, g, b)


def _occ_prune(x5, wocc):
    T, D, H, W, C = x5.shape
    xl = x5.reshape(T * D * H, W * C).astype(BF16)
    Q = _occ_mat(wocc, W, C).astype(BF16)
    occ = _pallas_dot(xl, Q)
    occ5 = occ.reshape(T, D, H, W)
    keep = (jax.nn.sigmoid(occ5) >= 0.5).astype(x5.dtype)[..., None]
    pruned = jax.lax.optimization_barrier(x5 * keep)
    return occ5, pruned


def kernel(x, p):
    _, Cin, T, D, H, W = x.shape
    x5 = x[0].transpose(1, 2, 3, 4, 0)

    stem = _conv_same(x5, _blockdiag_mats(p['stem_w'], W), p['stem_g'], p['stem_b'], 16)

    def enc(h, i, Co):
        h = _conv_down(h, p['enc%d_dw' % i], p['enc%d_dg' % i], p['enc%d_db' % i])
        S = h.shape[0]
        if S == 1:
            h = _conv_center(h, p['enc%d_rw' % i], p['enc%d_rg' % i], p['enc%d_rb' % i])
        else:
            h = _conv_same(h, _blockdiag_mats(p['enc%d_rw' % i], S),
                           p['enc%d_rg' % i], p['enc%d_rb' % i], Co)
        return h

    e1 = enc(stem, 1, 16)
    e2 = enc(e1, 2, 32)
    e3 = enc(e2, 3, 64)
    lat = enc(e3, 4, 128)

    def dec(h, skip, n, Co):
        h = _conv_up(h, p[n + '_uw'], p[n + '_ug'], p[n + '_ub'])
        h = jnp.concatenate([h, skip], axis=-1)
        S = h.shape[0]
        h = _conv_same(h, _blockdiag_mats(p[n + '_fw'], S), p[n + '_fg'], p[n + '_fb'], Co)
        return h

    d3 = dec(lat, e3, 'dec3', 64)
    occ3, d3 = _occ_prune(d3, p['occ3_w'])
    d2 = dec(d3, e2, 'dec2', 32)
    occ2, d2 = _occ_prune(d2, p['occ2_w'])
    d1 = dec(d2, e1, 'dec1', 16)
    occ1, d1 = _occ_prune(d1, p['occ1_w'])
    d0 = dec(d1, stem, 'dec0', 16)
    occ0, d0 = _occ_prune(d0, p['occ0_w'])

    Q3 = _occ_mat(p['off_w'], W, 16)
    off = _pallas_dot(d0.reshape(T * D * H, W * 16).astype(BF16), Q3.astype(BF16))
    offsets = jax.nn.sigmoid(off.reshape(T, D, H, W, 3)).transpose(4, 0, 1, 2, 3)[None]

    def to_out(o):
        return o[None, None]

    return offsets, to_out(occ3), to_out(occ2), to_out(occ1), to_out(occ0)
